# Initial kernel scaffold; baseline (speedup 1.0000x reference)
import jax
import jax.numpy as jnp
from jax.experimental import pallas as pl


def kernel(x, tables):
    raise NotImplementedError("write your pallas kernel here")


# speedup vs baseline: 1.2053x; 1.2053x over previous
import functools

import jax
import jax.numpy as jnp
from jax import lax
from jax.experimental import pallas as pl
from jax.experimental.pallas import tpu as pltpu
from jax.experimental.pallas import tpu_sc as plsc

_NUM_FIELDS = 26
_VOCAB = 100000
_EMBED_DIM = 32
_BATCH = 16384

_NC, _NS = 2, 16
_NW = _NC * _NS
_ROWS = _BATCH * _NUM_FIELDS
_CHUNK = _ROWS // _NW
_IW = 128
_IROWS = _CHUNK // _IW
_BLK_IROWS = 8
_BLK = _BLK_IROWS * _IW
_NBLK = _IROWS // _BLK_IROWS


def _mesh():
    return plsc.VectorSubcoreMesh(
        core_axis_name="c", subcore_axis_name="s",
        num_cores=_NC, num_subcores=_NS)


@functools.partial(
    pl.kernel,
    out_type=jax.ShapeDtypeStruct((_ROWS, _EMBED_DIM), jnp.float32),
    mesh=_mesh(),
    scratch_types=[
        pltpu.VMEM((_IROWS, _IW), jnp.int32),
        pltpu.VMEM((_IROWS, _IW), jnp.int32),
        pltpu.VMEM((_BLK, _EMBED_DIM), jnp.float32),
        pltpu.SemaphoreType.DMA,
    ],
    compiler_params=pltpu.CompilerParams(use_tc_tiling_on_sc=False),
)
def _embed_gather(x_hbm, off_hbm, tab_hbm, out_hbm, idx_v, off_v, rows_v, sem):
    wid = lax.axis_index("s") * _NC + lax.axis_index("c")
    irow0 = wid * _IROWS

    pltpu.sync_copy(x_hbm.at[pl.ds(irow0, _IROWS)], idx_v)
    pltpu.sync_copy(off_hbm, off_v)

    def add_offsets(r, carry):
        for c in range(_IW // 16):
            sl = pl.ds(c * 16, 16)
            idx_v[r, sl] = idx_v[r, sl] + off_v[r, sl]
        return carry

    lax.fori_loop(0, _IROWS, add_offsets, 0)

    def do_block(b, carry):
        handles = []
        for j in range(_BLK_IROWS):
            h = pltpu.async_copy(
                tab_hbm.at[idx_v.at[b * _BLK_IROWS + j]],
                rows_v.at[pl.ds(j * _IW, _IW)],
                sem)
            handles.append(h)
        for h in handles:
            h.wait()
        pltpu.sync_copy(
            rows_v, out_hbm.at[pl.ds(wid * _CHUNK + b * _BLK, _BLK)])
        return carry

    lax.fori_loop(0, _NBLK, do_block, 0)


def kernel(x, tables):
    x_flat = x.astype(jnp.int32).reshape(_IROWS * _NW, _IW)
    offsets = jnp.tile(
        jnp.arange(_NUM_FIELDS, dtype=jnp.int32) * _VOCAB,
        _ROWS // _NUM_FIELDS).reshape(_IROWS * _NW, _IW)
    off_chunk = offsets[:_IROWS]
    tab_flat = tables.reshape(_NUM_FIELDS * _VOCAB, _EMBED_DIM)
    out = _embed_gather(x_flat, off_chunk, tab_flat)
    return out.reshape(_BATCH, _NUM_FIELDS * _EMBED_DIM)


# speedup vs baseline: 4.3431x; 3.6034x over previous
import functools

import jax
import jax.numpy as jnp
from jax import lax
from jax.experimental import pallas as pl
from jax.experimental.pallas import tpu as pltpu
from jax.experimental.pallas import tpu_sc as plsc

_NUM_FIELDS = 26
_VOCAB = 100000
_EMBED_DIM = 32
_BATCH = 16384

_NC, _NS = 2, 16
_NW = _NC * _NS
_NROWS = _NUM_FIELDS * _EMBED_DIM
_RPW = _NROWS // _NW
_OCHUNK = _BATCH // 2


def _mesh():
    return plsc.VectorSubcoreMesh(
        core_axis_name="c", subcore_axis_name="s",
        num_cores=_NC, num_subcores=_NS)


@functools.partial(
    pl.kernel,
    out_type=jax.ShapeDtypeStruct((_NROWS, _BATCH), jnp.float32),
    mesh=_mesh(),
    scratch_types=[
        pltpu.VMEM((_VOCAB,), jnp.float32),
        pltpu.VMEM((_BATCH,), jnp.int32),
        pltpu.VMEM((_OCHUNK,), jnp.float32),
    ],
    compiler_params=pltpu.CompilerParams(
        use_tc_tiling_on_sc=True, needs_layout_passes=False),
)
def _embed_gather(x_hbm, tab_hbm, out_hbm, row_v, idx_v, out_v):
    wid = lax.axis_index("s") * _NC + lax.axis_index("c")
    r0 = wid * _RPW

    def do_row(k, prev_field):
        r = r0 + k
        field = lax.shift_right_logical(r, 5)

        @pl.when(jnp.logical_or(k == 0, field != prev_field))
        def _():
            pltpu.sync_copy(x_hbm.at[field], idx_v)

        pltpu.sync_copy(tab_hbm.at[r], row_v)

        for c in range(_BATCH // _OCHUNK):
            def gather16(g, carry):
                off = g * 16
                vals = plsc.load_gather(
                    row_v, [idx_v[pl.ds(c * _OCHUNK + off, 16)]])
                out_v[pl.ds(off, 16)] = vals
                return carry

            lax.fori_loop(0, _OCHUNK // 16, gather16, 0)
            pltpu.sync_copy(out_v, out_hbm.at[r, pl.ds(c * _OCHUNK, _OCHUNK)])
        return field

    lax.fori_loop(0, _RPW, do_row, jnp.int32(-1))


def kernel(x, tables):
    x_t = x.astype(jnp.int32).T
    tab_t = tables.transpose(0, 2, 1).reshape(_NROWS, _VOCAB)
    out_t = _embed_gather(x_t, tab_t)
    return out_t.T.reshape(_BATCH, _NUM_FIELDS * _EMBED_DIM)


# speedup vs baseline: 9.6937x; 2.2320x over previous
import functools

import jax
import jax.numpy as jnp
from jax import lax
from jax.experimental import pallas as pl
from jax.experimental.pallas import tpu as pltpu
from jax.experimental.pallas import tpu_sc as plsc

_NUM_FIELDS = 26
_VOCAB = 100000
_EMBED_DIM = 32
_BATCH = 16384

_NC, _NS = 2, 16
_NW = _NC * _NS
_NROWS = _NUM_FIELDS * _EMBED_DIM
_RPW = _NROWS // _NW
_OCHUNK = _BATCH // 2


def _mesh():
    return plsc.VectorSubcoreMesh(
        core_axis_name="c", subcore_axis_name="s",
        num_cores=_NC, num_subcores=_NS)


@functools.partial(
    pl.kernel,
    out_type=jax.ShapeDtypeStruct((_NROWS, _BATCH), jnp.float32),
    mesh=_mesh(),
    scratch_types=[
        pltpu.VMEM((_VOCAB,), jnp.float32),
        pltpu.VMEM((_BATCH,), jnp.int32),
        pltpu.VMEM((_OCHUNK,), jnp.float32),
    ],
    compiler_params=pltpu.CompilerParams(
        use_tc_tiling_on_sc=True, needs_layout_passes=False),
)
def _embed_gather(x_hbm, tab_hbm, out_hbm, row_v, idx_v, out_v):
    wid = lax.axis_index("s") * _NC + lax.axis_index("c")
    r0 = wid * _RPW

    def do_row(k, prev_field):
        r = r0 + k
        field = lax.shift_right_logical(r, 5)

        @pl.when(jnp.logical_or(k == 0, field != prev_field))
        def _():
            pltpu.sync_copy(x_hbm.at[field], idx_v)

        pltpu.sync_copy(tab_hbm.at[r], row_v)

        for c in range(_BATCH // _OCHUNK):
            @functools.partial(
                plsc.parallel_loop, 0, _OCHUNK // 16, unroll=8)
            def gather16(g):
                off = g * 16
                vals = plsc.load_gather(
                    row_v, [idx_v[pl.ds(c * _OCHUNK + off, 16)]])
                out_v[pl.ds(off, 16)] = vals

            pltpu.sync_copy(out_v, out_hbm.at[r, pl.ds(c * _OCHUNK, _OCHUNK)])
        return field

    lax.fori_loop(0, _RPW, do_row, jnp.int32(-1))


def kernel(x, tables):
    x_t = x.astype(jnp.int32).T
    tab_t = tables.transpose(0, 2, 1).reshape(_NROWS, _VOCAB)
    out_t = _embed_gather(x_t, tab_t)
    return out_t.T.reshape(_BATCH, _NUM_FIELDS * _EMBED_DIM)
